# baseline (speedup 1.0000x reference)
import functools

import jax
import jax.numpy as jnp
from jax import lax
from jax.experimental import pallas as pl
from jax.experimental.pallas import tpu as pltpu
from jax.experimental.pallas import tpu_sc as plsc

B, N, M, C, OUT, K = 2, 8192, 8192, 256, 128, 16
TMQ = 128
TM = 128

NC, NS = 2, 16
NW = NC * NS
R = B * M * K
RPW = R // NW
CHUNK = 128
NCH = RPW // CHUNK


def _table_body(lat_ref, pos_ref, wl_ref, wp_ref, t_ref):
    lat = lat_ref[0]
    p = pos_ref[0]
    t = lax.dot_general(lat, wl_ref[...], (((0,), (1,)), ((), ())),
                        preferred_element_type=jnp.float32)
    tp = lax.dot_general(p, wp_ref[...], (((0,), (1,)), ((), ())),
                         preferred_element_type=jnp.float32)
    t_ref[...] = (t - tp).astype(jnp.bfloat16)


def _build_table(latents, pos, w_lat, w_pos):
    return pl.pallas_call(
        _table_body,
        grid=(B,),
        in_specs=[
            pl.BlockSpec((1, C, N), lambda b: (b, 0, 0)),
            pl.BlockSpec((1, 3, N), lambda b: (b, 0, 0)),
            pl.BlockSpec((C, C), lambda b: (0, 0)),
            pl.BlockSpec((C, 3), lambda b: (0, 0)),
        ],
        out_specs=pl.BlockSpec((N, C), lambda b: (b, 0)),
        out_shape=jax.ShapeDtypeStruct((B * N, C), jnp.bfloat16),
    )(latents, pos, w_lat, w_pos)


NSEC = 32
SW = N // NSEC
CAP = 6
SENT = 3.0e38


def _knn_body(pos_ref, q_ref, idx_ref):
    b = pl.program_id(0)
    d2 = jnp.zeros((TMQ, N), jnp.float32)
    for c in range(3):
        qc = q_ref[0, :, c:c + 1]
        pc = pos_ref[0, c:c + 1, :]
        diff = qc - pc
        d2 = d2 + diff * diff
    lane = lax.broadcasted_iota(jnp.int32, (TMQ, N), 1) & (SW - 1)
    bits = lax.bitcast_convert_type(d2, jnp.int32)
    key = lax.bitcast_convert_type((bits & ~(SW - 1)) | lane, jnp.float32)

    secs = [key[:, s * SW:(s + 1) * SW] for s in range(NSEC)]
    cols = []
    for _ in range(CAP):
        for s in range(NSEC):
            sec = secs[s]
            mn = jnp.min(sec, axis=1, keepdims=True)
            secs[s] = jnp.where(sec == mn, SENT, sec)
            cols.append(mn)
    cands = jnp.concatenate(cols, axis=1)
    ncand = CAP * NSEC

    iota_c = lax.broadcasted_iota(jnp.int32, (TMQ, ncand), 1).astype(
        jnp.float32)
    out_cols = []
    for _ in range(K):
        w = jnp.min(cands, axis=1, keepdims=True)
        msk = cands == w
        p = jnp.min(jnp.where(msk, iota_c, jnp.float32(ncand)), axis=1,
                    keepdims=True).astype(jnp.int32)
        cands = jnp.where(msk, SENT, cands)
        wbits = lax.bitcast_convert_type(w, jnp.int32)
        lane_in = wbits & (SW - 1)
        sec = p % NSEC
        out_cols.append(sec * SW + lane_in)
    idx = jnp.concatenate(out_cols, axis=1)
    idx_ref[0] = idx + b * N


def _knn(pos, q_t):
    return pl.pallas_call(
        _knn_body,
        grid=(B, M // TMQ),
        in_specs=[
            pl.BlockSpec((1, 3, N), lambda b, i: (b, 0, 0)),
            pl.BlockSpec((1, TMQ, 3), lambda b, i: (b, i, 0)),
        ],
        out_specs=pl.BlockSpec((1, TMQ, K), lambda b, i: (b, i, 0)),
        out_shape=jax.ShapeDtypeStruct((B, M, K), jnp.int32),
    )(pos, q_t)


@functools.cache
def _make_gather():
    mesh = plsc.VectorSubcoreMesh(core_axis_name="c", subcore_axis_name="s")

    @functools.partial(
        pl.kernel,
        mesh=mesh,
        out_type=jax.ShapeDtypeStruct((R, C // 2), jnp.int32),
        scratch_types=[
            pltpu.VMEM((CHUNK,), jnp.int32),
            pltpu.VMEM((CHUNK, C // 2), jnp.int32),
            pltpu.SemaphoreType.DMA,
        ],
    )
    def gather(table_hbm, idx_hbm, out_hbm, idx_v, rows_v, sem):
        wid = lax.axis_index("s") * NC + lax.axis_index("c")

        def body(ch, carry):
            pltpu.sync_copy(idx_hbm.at[wid, ch], idx_v)
            pltpu.async_copy(table_hbm.at[idx_v], rows_v, sem).wait()
            pltpu.sync_copy(rows_v,
                            out_hbm.at[pl.ds(wid * RPW + ch * CHUNK, CHUNK)])
            return carry

        lax.fori_loop(0, NCH, body, 0)

    return gather


def _gather(table, idx3):
    return _make_gather()(table, idx3)


def _mlp_body(g_ref, q_ref, wp_ref, w1_ref, w2_ref, wo_ref,
              bin_ref, b1_ref, b2_ref, bo_ref, o_ref):
    q = q_ref[0]
    qc = lax.dot_general(q, wp_ref[...], (((1,), (1,)), ((), ())),
                         preferred_element_type=jnp.float32)
    qc = qc + bin_ref[...]
    g = g_ref[...].astype(jnp.float32)
    x = g.reshape(TM, K, C) + qc[:, None, :]
    x = x.reshape(TM * K, C)
    h = lax.dot_general(jnp.maximum(x, 0.0), w1_ref[...],
                        (((1,), (1,)), ((), ())),
                        preferred_element_type=jnp.float32) + b1_ref[...]
    h = lax.dot_general(jnp.maximum(h, 0.0), w2_ref[...],
                        (((1,), (1,)), ((), ())),
                        preferred_element_type=jnp.float32) + b2_ref[...]
    y = jnp.max(h.reshape(TM, K, C), axis=1)
    o = lax.dot_general(y, wo_ref[...], (((1,), (1,)), ((), ())),
                        preferred_element_type=jnp.float32) + bo_ref[...]
    o_ref[0] = o


def _mlp(g, q_t, w_pos, w1, w2, w_out, b_in, b1, b2, b_out):
    nmt = M // TM
    return pl.pallas_call(
        _mlp_body,
        grid=(B, nmt),
        in_specs=[
            pl.BlockSpec((TM * K, C), lambda b, i: (b * nmt + i, 0)),
            pl.BlockSpec((1, TM, 3), lambda b, i: (b, i, 0)),
            pl.BlockSpec((C, 3), lambda b, i: (0, 0)),
            pl.BlockSpec((C, C), lambda b, i: (0, 0)),
            pl.BlockSpec((C, C), lambda b, i: (0, 0)),
            pl.BlockSpec((OUT, C), lambda b, i: (0, 0)),
            pl.BlockSpec((1, C), lambda b, i: (0, 0)),
            pl.BlockSpec((1, C), lambda b, i: (0, 0)),
            pl.BlockSpec((1, C), lambda b, i: (0, 0)),
            pl.BlockSpec((1, OUT), lambda b, i: (0, 0)),
        ],
        out_specs=pl.BlockSpec((1, TM, OUT), lambda b, i: (b, i, 0)),
        out_shape=jax.ShapeDtypeStruct((B, M, OUT), jnp.float32),
    )(g, q_t, w_pos, w1, w2, w_out, b_in, b1, b2, b_out)


def kernel(pos, pos_non_manifold, latents, W_in, b_in, W1, b1, W2, b2,
           W_out, b_out):
    w_lat = W_in[:, :C]
    w_pos = W_in[:, C:]
    q_t = jnp.swapaxes(pos_non_manifold, 1, 2)

    table = _build_table(latents, pos, w_lat, w_pos)
    table32 = lax.bitcast_convert_type(
        table.reshape(B * N, C // 2, 2), jnp.int32)
    idx = _knn(pos, q_t)
    idx3 = idx.reshape(NW, NCH, CHUNK)
    g32 = _gather(table32, idx3)
    g = lax.bitcast_convert_type(g32, jnp.bfloat16).reshape(R, C)

    out_t = _mlp(g, q_t, w_pos, W1, W2, W_out,
                 b_in.reshape(1, C), b1.reshape(1, C), b2.reshape(1, C),
                 b_out.reshape(1, OUT))
    return jnp.swapaxes(out_t, 1, 2)


# speedup vs baseline: 1.9643x; 1.9643x over previous
import functools

import jax
import jax.numpy as jnp
from jax import lax
from jax.experimental import pallas as pl
from jax.experimental.pallas import tpu as pltpu
from jax.experimental.pallas import tpu_sc as plsc

B, N, M, C, OUT, K = 2, 8192, 8192, 256, 128, 16
TMQ = 128
TM = 128

NC, NS = 2, 16
NW = NC * NS
R = B * M * K
RPW = R // NW
CHUNK = 128
NCH = RPW // CHUNK


def _table_body(lat_ref, pos_ref, wl_ref, wp_ref, t_ref):
    lat = lat_ref[0]
    p = pos_ref[0]
    t = lax.dot_general(lat, wl_ref[...], (((0,), (1,)), ((), ())),
                        preferred_element_type=jnp.float32)
    tp = lax.dot_general(p, wp_ref[...], (((0,), (1,)), ((), ())),
                         preferred_element_type=jnp.float32)
    t = t - tp
    lo = lax.bitcast_convert_type(t[:, :C // 2], jnp.int32)
    hi = lax.bitcast_convert_type(t[:, C // 2:], jnp.int32)
    lo = lo + 0x7FFF + (lax.shift_right_logical(lo, 16) & 1)
    hi = hi + 0x7FFF + (lax.shift_right_logical(hi, 16) & 1)
    t_ref[...] = (hi & jnp.int32(-65536)) | lax.shift_right_logical(lo, 16)


def _build_table(latents, pos, w_lat, w_pos):
    return pl.pallas_call(
        _table_body,
        grid=(B,),
        in_specs=[
            pl.BlockSpec((1, C, N), lambda b: (b, 0, 0)),
            pl.BlockSpec((1, 3, N), lambda b: (b, 0, 0)),
            pl.BlockSpec((C, C), lambda b: (0, 0)),
            pl.BlockSpec((C, 3), lambda b: (0, 0)),
        ],
        out_specs=pl.BlockSpec((N, C // 2), lambda b: (b, 0)),
        out_shape=jax.ShapeDtypeStruct((B * N, C // 2), jnp.int32),
    )(latents, pos, w_lat, w_pos)


NSEC = 32
SW = N // NSEC
CAP = 6
SENT = 3.0e38


def _knn_body(pos_ref, q_ref, idx_ref):
    b = pl.program_id(0)
    d2 = jnp.zeros((TMQ, N), jnp.float32)
    for c in range(3):
        qc = q_ref[0, :, c:c + 1]
        pc = pos_ref[0, c:c + 1, :]
        diff = qc - pc
        d2 = d2 + diff * diff
    lane = lax.broadcasted_iota(jnp.int32, (TMQ, N), 1) & (SW - 1)
    bits = lax.bitcast_convert_type(d2, jnp.int32)
    key = lax.bitcast_convert_type((bits & ~(SW - 1)) | lane, jnp.float32)

    secs = [key[:, s * SW:(s + 1) * SW] for s in range(NSEC)]
    cols = []
    for _ in range(CAP):
        for s in range(NSEC):
            sec = secs[s]
            mn = jnp.min(sec, axis=1, keepdims=True)
            secs[s] = jnp.where(sec == mn, SENT, sec)
            cols.append(mn)
    cands = jnp.concatenate(cols, axis=1)
    ncand = CAP * NSEC

    iota_c = lax.broadcasted_iota(jnp.int32, (TMQ, ncand), 1).astype(
        jnp.float32)
    out_cols = []
    for _ in range(K):
        w = jnp.min(cands, axis=1, keepdims=True)
        msk = cands == w
        p = jnp.min(jnp.where(msk, iota_c, jnp.float32(ncand)), axis=1,
                    keepdims=True).astype(jnp.int32)
        cands = jnp.where(msk, SENT, cands)
        wbits = lax.bitcast_convert_type(w, jnp.int32)
        lane_in = wbits & (SW - 1)
        sec = p % NSEC
        out_cols.append(sec * SW + lane_in)
    idx = jnp.concatenate(out_cols, axis=1)
    idx_ref[0] = idx + b * N


def _knn(pos, q_t):
    return pl.pallas_call(
        _knn_body,
        grid=(B, M // TMQ),
        in_specs=[
            pl.BlockSpec((1, 3, N), lambda b, i: (b, 0, 0)),
            pl.BlockSpec((1, TMQ, 3), lambda b, i: (b, i, 0)),
        ],
        out_specs=pl.BlockSpec((1, TMQ, K), lambda b, i: (b, i, 0)),
        out_shape=jax.ShapeDtypeStruct((B, M, K), jnp.int32),
    )(pos, q_t)


@functools.cache
def _make_gather():
    mesh = plsc.VectorSubcoreMesh(core_axis_name="c", subcore_axis_name="s")

    @functools.partial(
        pl.kernel,
        mesh=mesh,
        out_type=jax.ShapeDtypeStruct((R, C // 2), jnp.int32),
        scratch_types=[
            pltpu.VMEM((CHUNK,), jnp.int32),
            pltpu.VMEM((CHUNK, C // 2), jnp.int32),
            pltpu.SemaphoreType.DMA,
        ],
    )
    def gather(table_hbm, idx_hbm, out_hbm, idx_v, rows_v, sem):
        wid = lax.axis_index("s") * NC + lax.axis_index("c")

        def body(ch, carry):
            pltpu.sync_copy(idx_hbm.at[wid, ch], idx_v)
            pltpu.async_copy(table_hbm.at[idx_v], rows_v, sem).wait()
            pltpu.sync_copy(rows_v,
                            out_hbm.at[pl.ds(wid * RPW + ch * CHUNK, CHUNK)])
            return carry

        lax.fori_loop(0, NCH, body, 0)

    return gather


def _gather(table, idx3):
    return _make_gather()(table, idx3)


def _mlp_body(g_ref, q_ref, wp_ref, w1_ref, w2_ref, wo_ref,
              bin_ref, b1_ref, b2_ref, bo_ref, o_ref):
    q = q_ref[0]
    qc = lax.dot_general(q, wp_ref[...], (((1,), (1,)), ((), ())),
                         preferred_element_type=jnp.float32)
    qc = qc + bin_ref[...]
    g32 = g_ref[...]
    g_lo = lax.bitcast_convert_type(lax.shift_left(g32, 16), jnp.float32)
    g_hi = lax.bitcast_convert_type(g32 & jnp.int32(-65536), jnp.float32)
    g = jnp.concatenate([g_lo, g_hi], axis=1)
    x = g.reshape(TM, K, C) + qc[:, None, :]
    x = x.reshape(TM * K, C)
    h = lax.dot_general(jnp.maximum(x, 0.0), w1_ref[...],
                        (((1,), (1,)), ((), ())),
                        preferred_element_type=jnp.float32) + b1_ref[...]
    h = lax.dot_general(jnp.maximum(h, 0.0), w2_ref[...],
                        (((1,), (1,)), ((), ())),
                        preferred_element_type=jnp.float32) + b2_ref[...]
    y = jnp.max(h.reshape(TM, K, C), axis=1)
    o = lax.dot_general(y, wo_ref[...], (((1,), (1,)), ((), ())),
                        preferred_element_type=jnp.float32) + bo_ref[...]
    o_ref[0] = o


def _mlp(g, q_t, w_pos, w1, w2, w_out, b_in, b1, b2, b_out):
    nmt = M // TM
    return pl.pallas_call(
        _mlp_body,
        grid=(B, nmt),
        in_specs=[
            pl.BlockSpec((TM * K, C // 2), lambda b, i: (b * nmt + i, 0)),
            pl.BlockSpec((1, TM, 3), lambda b, i: (b, i, 0)),
            pl.BlockSpec((C, 3), lambda b, i: (0, 0)),
            pl.BlockSpec((C, C), lambda b, i: (0, 0)),
            pl.BlockSpec((C, C), lambda b, i: (0, 0)),
            pl.BlockSpec((OUT, C), lambda b, i: (0, 0)),
            pl.BlockSpec((1, C), lambda b, i: (0, 0)),
            pl.BlockSpec((1, C), lambda b, i: (0, 0)),
            pl.BlockSpec((1, C), lambda b, i: (0, 0)),
            pl.BlockSpec((1, OUT), lambda b, i: (0, 0)),
        ],
        out_specs=pl.BlockSpec((1, TM, OUT), lambda b, i: (b, i, 0)),
        out_shape=jax.ShapeDtypeStruct((B, M, OUT), jnp.float32),
    )(g, q_t, w_pos, w1, w2, w_out, b_in, b1, b2, b_out)


def kernel(pos, pos_non_manifold, latents, W_in, b_in, W1, b1, W2, b2,
           W_out, b_out):
    w_lat = W_in[:, :C]
    w_pos = W_in[:, C:]
    q_t = jnp.swapaxes(pos_non_manifold, 1, 2)

    table = _build_table(latents, pos, w_lat, w_pos)
    idx = _knn(pos, q_t)
    idx3 = idx.reshape(NW, NCH, CHUNK)
    g = _gather(table, idx3)

    out_t = _mlp(g, q_t, w_pos, W1, W2, W_out,
                 b_in.reshape(1, C), b1.reshape(1, C), b2.reshape(1, C),
                 b_out.reshape(1, OUT))
    return jnp.swapaxes(out_t, 1, 2)


# speedup vs baseline: 2.0657x; 1.0516x over previous
import functools

import jax
import jax.numpy as jnp
from jax import lax
from jax.experimental import pallas as pl
from jax.experimental.pallas import tpu as pltpu
from jax.experimental.pallas import tpu_sc as plsc

B, N, M, C, OUT, K = 2, 8192, 8192, 256, 128, 16
TMQ = 256
TM = 128

NC, NS = 2, 16
NW = NC * NS
R = B * M * K
RPW = R // NW
CHUNK = 128
NCH = RPW // CHUNK


def _table_body(lat_ref, pos_ref, wl_ref, wp_ref, t_ref):
    lat = lat_ref[0]
    p = pos_ref[0]
    t = lax.dot_general(lat, wl_ref[...], (((0,), (1,)), ((), ())),
                        preferred_element_type=jnp.float32)
    tp = lax.dot_general(p, wp_ref[...], (((0,), (1,)), ((), ())),
                         preferred_element_type=jnp.float32)
    t = t - tp
    lo = lax.bitcast_convert_type(t[:, :C // 2], jnp.int32)
    hi = lax.bitcast_convert_type(t[:, C // 2:], jnp.int32)
    lo = lo + 0x7FFF + (lax.shift_right_logical(lo, 16) & 1)
    hi = hi + 0x7FFF + (lax.shift_right_logical(hi, 16) & 1)
    t_ref[...] = (hi & jnp.int32(-65536)) | lax.shift_right_logical(lo, 16)


def _build_table(latents, pos, w_lat, w_pos):
    return pl.pallas_call(
        _table_body,
        grid=(B,),
        in_specs=[
            pl.BlockSpec((1, C, N), lambda b: (b, 0, 0)),
            pl.BlockSpec((1, 3, N), lambda b: (b, 0, 0)),
            pl.BlockSpec((C, C), lambda b: (0, 0)),
            pl.BlockSpec((C, 3), lambda b: (0, 0)),
        ],
        out_specs=pl.BlockSpec((N, C // 2), lambda b: (b, 0)),
        out_shape=jax.ShapeDtypeStruct((B * N, C // 2), jnp.int32),
    )(latents, pos, w_lat, w_pos)


NSEC = 32
SW = N // NSEC
CAP = 6
SENT = 3.0e38


def _knn_body(pos_ref, q_ref, idx_ref):
    b = pl.program_id(0)
    d2 = jnp.zeros((TMQ, N), jnp.float32)
    for c in range(3):
        qc = q_ref[0, :, c:c + 1]
        pc = pos_ref[0, c:c + 1, :]
        diff = qc - pc
        d2 = d2 + diff * diff
    lane = lax.broadcasted_iota(jnp.int32, (TMQ, N), 1) & (SW - 1)
    bits = lax.bitcast_convert_type(d2, jnp.int32)
    key = lax.bitcast_convert_type((bits & ~(SW - 1)) | lane, jnp.float32)

    secs = [key[:, s * SW:(s + 1) * SW] for s in range(NSEC)]
    cols = []
    for _ in range(CAP):
        for s in range(NSEC):
            sec = secs[s]
            mn = jnp.min(sec, axis=1, keepdims=True)
            secs[s] = jnp.where(sec == mn, SENT, sec)
            cols.append(mn)
    cands = jnp.concatenate(cols, axis=1)
    ncand = CAP * NSEC

    iota_c = lax.broadcasted_iota(jnp.int32, (TMQ, ncand), 1).astype(
        jnp.float32)
    out_cols = []
    for _ in range(K):
        w = jnp.min(cands, axis=1, keepdims=True)
        msk = cands == w
        p = jnp.min(jnp.where(msk, iota_c, jnp.float32(ncand)), axis=1,
                    keepdims=True).astype(jnp.int32)
        cands = jnp.where(msk, SENT, cands)
        wbits = lax.bitcast_convert_type(w, jnp.int32)
        lane_in = wbits & (SW - 1)
        sec = p % NSEC
        out_cols.append(sec * SW + lane_in)
    idx = jnp.concatenate(out_cols, axis=1)
    idx_ref[0] = idx + b * N


def _knn(pos, q_t):
    return pl.pallas_call(
        _knn_body,
        grid=(B, M // TMQ),
        in_specs=[
            pl.BlockSpec((1, 3, N), lambda b, i: (b, 0, 0)),
            pl.BlockSpec((1, TMQ, 3), lambda b, i: (b, i, 0)),
        ],
        out_specs=pl.BlockSpec((1, TMQ, K), lambda b, i: (b, i, 0)),
        out_shape=jax.ShapeDtypeStruct((B, M, K), jnp.int32),
    )(pos, q_t)


@functools.cache
def _make_gather():
    mesh = plsc.VectorSubcoreMesh(core_axis_name="c", subcore_axis_name="s")

    @functools.partial(
        pl.kernel,
        mesh=mesh,
        out_type=jax.ShapeDtypeStruct((R, C // 2), jnp.int32),
        scratch_types=[
            pltpu.VMEM((CHUNK,), jnp.int32),
            pltpu.VMEM((CHUNK, C // 2), jnp.int32),
            pltpu.SemaphoreType.DMA,
        ],
    )
    def gather(table_hbm, idx_hbm, out_hbm, idx_v, rows_v, sem):
        wid = lax.axis_index("s") * NC + lax.axis_index("c")

        def body(ch, carry):
            pltpu.sync_copy(idx_hbm.at[wid, ch], idx_v)
            pltpu.async_copy(table_hbm.at[idx_v], rows_v, sem).wait()
            pltpu.sync_copy(rows_v,
                            out_hbm.at[pl.ds(wid * RPW + ch * CHUNK, CHUNK)])
            return carry

        lax.fori_loop(0, NCH, body, 0)

    return gather


def _gather(table, idx3):
    return _make_gather()(table, idx3)


def _mlp_body(g_ref, q_ref, wp_ref, w1_ref, w2_ref, wo_ref,
              bin_ref, b1_ref, b2_ref, bo_ref, o_ref):
    q = q_ref[0]
    qc = lax.dot_general(q, wp_ref[...], (((1,), (1,)), ((), ())),
                         preferred_element_type=jnp.float32)
    qc = qc + bin_ref[...]
    g32 = g_ref[...]
    g_lo = lax.bitcast_convert_type(lax.shift_left(g32, 16), jnp.float32)
    g_hi = lax.bitcast_convert_type(g32 & jnp.int32(-65536), jnp.float32)
    g = jnp.concatenate([g_lo, g_hi], axis=1)
    x = g.reshape(TM, K, C) + qc[:, None, :]
    x = x.reshape(TM * K, C)
    h = lax.dot_general(jnp.maximum(x, 0.0), w1_ref[...],
                        (((1,), (1,)), ((), ())),
                        preferred_element_type=jnp.float32) + b1_ref[...]
    h = lax.dot_general(jnp.maximum(h, 0.0), w2_ref[...],
                        (((1,), (1,)), ((), ())),
                        preferred_element_type=jnp.float32) + b2_ref[...]
    y = jnp.max(h.reshape(TM, K, C), axis=1)
    o = lax.dot_general(y, wo_ref[...], (((1,), (1,)), ((), ())),
                        preferred_element_type=jnp.float32) + bo_ref[...]
    o_ref[0] = o


def _mlp(g, q_t, w_pos, w1, w2, w_out, b_in, b1, b2, b_out):
    nmt = M // TM
    return pl.pallas_call(
        _mlp_body,
        grid=(B, nmt),
        in_specs=[
            pl.BlockSpec((TM * K, C // 2), lambda b, i: (b * nmt + i, 0)),
            pl.BlockSpec((1, TM, 3), lambda b, i: (b, i, 0)),
            pl.BlockSpec((C, 3), lambda b, i: (0, 0)),
            pl.BlockSpec((C, C), lambda b, i: (0, 0)),
            pl.BlockSpec((C, C), lambda b, i: (0, 0)),
            pl.BlockSpec((OUT, C), lambda b, i: (0, 0)),
            pl.BlockSpec((1, C), lambda b, i: (0, 0)),
            pl.BlockSpec((1, C), lambda b, i: (0, 0)),
            pl.BlockSpec((1, C), lambda b, i: (0, 0)),
            pl.BlockSpec((1, OUT), lambda b, i: (0, 0)),
        ],
        out_specs=pl.BlockSpec((1, TM, OUT), lambda b, i: (b, i, 0)),
        out_shape=jax.ShapeDtypeStruct((B, M, OUT), jnp.float32),
    )(g, q_t, w_pos, w1, w2, w_out, b_in, b1, b2, b_out)


def kernel(pos, pos_non_manifold, latents, W_in, b_in, W1, b1, W2, b2,
           W_out, b_out):
    w_lat = W_in[:, :C]
    w_pos = W_in[:, C:]
    q_t = jnp.swapaxes(pos_non_manifold, 1, 2)

    table = _build_table(latents, pos, w_lat, w_pos)
    idx = _knn(pos, q_t)
    idx3 = idx.reshape(NW, NCH, CHUNK)
    g = _gather(table, idx3)

    out_t = _mlp(g, q_t, w_pos, W1, W2, W_out,
                 b_in.reshape(1, C), b1.reshape(1, C), b2.reshape(1, C),
                 b_out.reshape(1, OUT))
    return jnp.swapaxes(out_t, 1, 2)


# speedup vs baseline: 2.3225x; 1.1243x over previous
import functools

import jax
import jax.numpy as jnp
from jax import lax
from jax.experimental import pallas as pl
from jax.experimental.pallas import tpu as pltpu
from jax.experimental.pallas import tpu_sc as plsc

B, N, M, C, OUT, K = 2, 8192, 8192, 256, 128, 16
TMQ = 256
TM = 128

NC, NS = 2, 16
NW = NC * NS
R = B * M * K
RPW = R // NW
CHUNK = 128
NCH = RPW // CHUNK


def _table_body(lat_ref, pos_ref, wl_ref, wp_ref, t_ref):
    lat = lat_ref[0]
    p = pos_ref[0]
    t = lax.dot_general(lat, wl_ref[...], (((0,), (1,)), ((), ())),
                        preferred_element_type=jnp.float32)
    tp = lax.dot_general(p, wp_ref[...], (((0,), (1,)), ((), ())),
                         preferred_element_type=jnp.float32)
    t = t - tp
    lo = lax.bitcast_convert_type(t[:, :C // 2], jnp.int32)
    hi = lax.bitcast_convert_type(t[:, C // 2:], jnp.int32)
    lo = lo + 0x7FFF + (lax.shift_right_logical(lo, 16) & 1)
    hi = hi + 0x7FFF + (lax.shift_right_logical(hi, 16) & 1)
    t_ref[...] = (hi & jnp.int32(-65536)) | lax.shift_right_logical(lo, 16)


def _build_table(latents, pos, w_lat, w_pos):
    nb = latents.shape[0]
    return pl.pallas_call(
        _table_body,
        grid=(nb,),
        in_specs=[
            pl.BlockSpec((1, C, N), lambda b: (b, 0, 0)),
            pl.BlockSpec((1, 3, N), lambda b: (b, 0, 0)),
            pl.BlockSpec((C, C), lambda b: (0, 0)),
            pl.BlockSpec((C, 3), lambda b: (0, 0)),
        ],
        out_specs=pl.BlockSpec((N, C // 2), lambda b: (b, 0)),
        out_shape=jax.ShapeDtypeStruct((nb * N, C // 2), jnp.int32),
    )(latents, pos, w_lat, w_pos)


NSEC = 32
SW = N // NSEC
CAP = 6
SENT = 3.0e38


def _knn_body(pos_ref, q_ref, idx_ref):
    b = pl.program_id(0)
    d2 = jnp.zeros((TMQ, N), jnp.float32)
    for c in range(3):
        qc = q_ref[0, :, c:c + 1]
        pc = pos_ref[0, c:c + 1, :]
        diff = qc - pc
        d2 = d2 + diff * diff
    lane = lax.broadcasted_iota(jnp.int32, (TMQ, N), 1) & (SW - 1)
    bits = lax.bitcast_convert_type(d2, jnp.int32)
    key = lax.bitcast_convert_type((bits & ~(SW - 1)) | lane, jnp.float32)

    secs = [key[:, s * SW:(s + 1) * SW] for s in range(NSEC)]
    cols = []
    for _ in range(CAP):
        for s in range(NSEC):
            sec = secs[s]
            mn = jnp.min(sec, axis=1, keepdims=True)
            secs[s] = jnp.where(sec == mn, SENT, sec)
            cols.append(mn)
    cands = jnp.concatenate(cols, axis=1)
    ncand = CAP * NSEC

    iota_c = lax.broadcasted_iota(jnp.int32, (TMQ, ncand), 1).astype(
        jnp.float32)
    out_cols = []
    for _ in range(K):
        w = jnp.min(cands, axis=1, keepdims=True)
        msk = cands == w
        p = jnp.min(jnp.where(msk, iota_c, jnp.float32(ncand)), axis=1,
                    keepdims=True).astype(jnp.int32)
        cands = jnp.where(msk, SENT, cands)
        wbits = lax.bitcast_convert_type(w, jnp.int32)
        lane_in = wbits & (SW - 1)
        sec = p % NSEC
        out_cols.append(sec * SW + lane_in)
    idx = jnp.concatenate(out_cols, axis=1)
    idx_ref[0] = idx + b * N


def _knn(pos, q_t):
    nb = pos.shape[0]
    return pl.pallas_call(
        _knn_body,
        grid=(nb, M // TMQ),
        in_specs=[
            pl.BlockSpec((1, 3, N), lambda b, i: (b, 0, 0)),
            pl.BlockSpec((1, TMQ, 3), lambda b, i: (b, i, 0)),
        ],
        out_specs=pl.BlockSpec((1, TMQ, K), lambda b, i: (b, i, 0)),
        out_shape=jax.ShapeDtypeStruct((nb, M, K), jnp.int32),
    )(pos, q_t)


@functools.cache
def _make_gather(rows):
    mesh = plsc.VectorSubcoreMesh(core_axis_name="c", subcore_axis_name="s")
    rpw = rows // NW
    nch = rpw // CHUNK

    @functools.partial(
        pl.kernel,
        mesh=mesh,
        out_type=jax.ShapeDtypeStruct((rows, C // 2), jnp.int32),
        scratch_types=[
            pltpu.VMEM((CHUNK,), jnp.int32),
            pltpu.VMEM((CHUNK, C // 2), jnp.int32),
            pltpu.SemaphoreType.DMA,
        ],
    )
    def gather(table_hbm, idx_hbm, out_hbm, idx_v, rows_v, sem):
        wid = lax.axis_index("s") * NC + lax.axis_index("c")

        def body(ch, carry):
            pltpu.sync_copy(idx_hbm.at[wid, ch], idx_v)
            pltpu.async_copy(table_hbm.at[idx_v], rows_v, sem).wait()
            pltpu.sync_copy(rows_v,
                            out_hbm.at[pl.ds(wid * rpw + ch * CHUNK, CHUNK)])
            return carry

        lax.fori_loop(0, nch, body, 0)

    return gather


def _gather(table, idx3):
    rows = idx3.shape[0] * idx3.shape[1] * idx3.shape[2]
    return _make_gather(rows)(table, idx3)


def _mlp_body(g_ref, q_ref, wp_ref, w1_ref, w2_ref, wo_ref,
              bin_ref, b1_ref, b2_ref, bo_ref, o_ref):
    q = q_ref[0]
    qc = lax.dot_general(q, wp_ref[...], (((1,), (1,)), ((), ())),
                         preferred_element_type=jnp.float32)
    qc = qc + bin_ref[...]
    g32 = g_ref[...]
    g_lo = lax.bitcast_convert_type(lax.shift_left(g32, 16), jnp.float32)
    g_hi = lax.bitcast_convert_type(g32 & jnp.int32(-65536), jnp.float32)
    g = jnp.concatenate([g_lo, g_hi], axis=1)
    x = g.reshape(TM, K, C) + qc[:, None, :]
    x = x.reshape(TM * K, C)
    h = lax.dot_general(jnp.maximum(x, 0.0), w1_ref[...],
                        (((1,), (1,)), ((), ())),
                        preferred_element_type=jnp.float32) + b1_ref[...]
    h = lax.dot_general(jnp.maximum(h, 0.0), w2_ref[...],
                        (((1,), (1,)), ((), ())),
                        preferred_element_type=jnp.float32) + b2_ref[...]
    y = jnp.max(h.reshape(TM, K, C), axis=1)
    o = lax.dot_general(y, wo_ref[...], (((1,), (1,)), ((), ())),
                        preferred_element_type=jnp.float32) + bo_ref[...]
    o_ref[0] = o


def _mlp(g, q_t, w_pos, w1, w2, w_out, b_in, b1, b2, b_out):
    nmt = M // TM
    nb = q_t.shape[0]
    return pl.pallas_call(
        _mlp_body,
        grid=(nb, nmt),
        in_specs=[
            pl.BlockSpec((TM * K, C // 2), lambda b, i: (b * nmt + i, 0)),
            pl.BlockSpec((1, TM, 3), lambda b, i: (b, i, 0)),
            pl.BlockSpec((C, 3), lambda b, i: (0, 0)),
            pl.BlockSpec((C, C), lambda b, i: (0, 0)),
            pl.BlockSpec((C, C), lambda b, i: (0, 0)),
            pl.BlockSpec((OUT, C), lambda b, i: (0, 0)),
            pl.BlockSpec((1, C), lambda b, i: (0, 0)),
            pl.BlockSpec((1, C), lambda b, i: (0, 0)),
            pl.BlockSpec((1, C), lambda b, i: (0, 0)),
            pl.BlockSpec((1, OUT), lambda b, i: (0, 0)),
        ],
        out_specs=pl.BlockSpec((1, TM, OUT), lambda b, i: (b, i, 0)),
        out_shape=jax.ShapeDtypeStruct((nb, M, OUT), jnp.float32),
    )(g, q_t, w_pos, w1, w2, w_out, b_in, b1, b2, b_out)


def kernel(pos, pos_non_manifold, latents, W_in, b_in, W1, b1, W2, b2,
           W_out, b_out):
    w_lat = W_in[:, :C]
    w_pos = W_in[:, C:]
    q_t = jnp.swapaxes(pos_non_manifold, 1, 2)

    outs = []
    for b in range(B):
        pos_b = pos[b:b + 1]
        q_b = q_t[b:b + 1]
        table_b = _build_table(latents[b:b + 1], pos_b, w_lat, w_pos)
        idx_b = _knn(pos_b, q_b)
        idx3_b = idx_b.reshape(NW, (M * K) // (NW * CHUNK), CHUNK)
        g_b = _gather(table_b, idx3_b)
        outs.append(_mlp(g_b, q_b, w_pos, W1, W2, W_out,
                         b_in.reshape(1, C), b1.reshape(1, C),
                         b2.reshape(1, C), b_out.reshape(1, OUT)))
    out_t = jnp.concatenate(outs, axis=0)
    return jnp.swapaxes(out_t, 1, 2)


# speedup vs baseline: 2.3229x; 1.0002x over previous
import functools

import jax
import jax.numpy as jnp
from jax import lax
from jax.experimental import pallas as pl
from jax.experimental.pallas import tpu as pltpu
from jax.experimental.pallas import tpu_sc as plsc

B, N, M, C, OUT, K = 2, 8192, 8192, 256, 128, 16
TMQ = 256
TM = 128

NC, NS = 2, 16
NW = NC * NS
CHUNK = 128


def _table_body(lat_ref, pos_ref, wl_ref, wp_ref, t_ref):
    lat = lat_ref[0]
    p = pos_ref[0]
    t = lax.dot_general(lat, wl_ref[...], (((0,), (1,)), ((), ())),
                        preferred_element_type=jnp.float32)
    tp = lax.dot_general(p, wp_ref[...], (((0,), (1,)), ((), ())),
                         preferred_element_type=jnp.float32)
    t = t - tp
    lo = lax.bitcast_convert_type(t[:, :C // 2], jnp.int32)
    hi = lax.bitcast_convert_type(t[:, C // 2:], jnp.int32)
    lo = lo + 0x7FFF + (lax.shift_right_logical(lo, 16) & 1)
    hi = hi + 0x7FFF + (lax.shift_right_logical(hi, 16) & 1)
    t_ref[...] = (hi & jnp.int32(-65536)) | lax.shift_right_logical(lo, 16)


def _build_table(latents, pos, w_lat, w_pos):
    nb = latents.shape[0]
    return pl.pallas_call(
        _table_body,
        grid=(nb,),
        in_specs=[
            pl.BlockSpec((1, C, N), lambda b: (b, 0, 0)),
            pl.BlockSpec((1, 3, N), lambda b: (b, 0, 0)),
            pl.BlockSpec((C, C), lambda b: (0, 0)),
            pl.BlockSpec((C, 3), lambda b: (0, 0)),
        ],
        out_specs=pl.BlockSpec((N, C // 2), lambda b: (b, 0)),
        out_shape=jax.ShapeDtypeStruct((nb * N, C // 2), jnp.int32),
    )(latents, pos, w_lat, w_pos)


NSEC = 32
SW = N // NSEC
CAP = 6
SENT = 3.0e38


def _knn_body(pos_ref, q_ref, idx_ref):
    b = pl.program_id(0)
    d2 = jnp.zeros((TMQ, N), jnp.float32)
    for c in range(3):
        qc = q_ref[0, :, c:c + 1]
        pc = pos_ref[0, c:c + 1, :]
        diff = qc - pc
        d2 = d2 + diff * diff
    lane = lax.broadcasted_iota(jnp.int32, (TMQ, N), 1) & (SW - 1)
    bits = lax.bitcast_convert_type(d2, jnp.int32)
    key = lax.bitcast_convert_type((bits & ~(SW - 1)) | lane, jnp.float32)

    secs = [key[:, s * SW:(s + 1) * SW] for s in range(NSEC)]
    cols = []
    for _ in range(CAP):
        for s in range(NSEC):
            sec = secs[s]
            mn = jnp.min(sec, axis=1, keepdims=True)
            secs[s] = jnp.where(sec == mn, SENT, sec)
            cols.append(mn)
    cands = jnp.concatenate(cols, axis=1)
    ncand = CAP * NSEC

    iota_c = lax.broadcasted_iota(jnp.int32, (TMQ, ncand), 1).astype(
        jnp.float32)
    out_cols = []
    for _ in range(K):
        w = jnp.min(cands, axis=1, keepdims=True)
        msk = cands == w
        p = jnp.min(jnp.where(msk, iota_c, jnp.float32(ncand)), axis=1,
                    keepdims=True).astype(jnp.int32)
        cands = jnp.where(msk, SENT, cands)
        wbits = lax.bitcast_convert_type(w, jnp.int32)
        lane_in = wbits & (SW - 1)
        sec = p % NSEC
        out_cols.append(sec * SW + lane_in)
    idx = jnp.concatenate(out_cols, axis=1)
    idx_ref[0] = idx + b * N


def _knn(pos, q_t):
    nb = pos.shape[0]
    return pl.pallas_call(
        _knn_body,
        grid=(nb, M // TMQ),
        in_specs=[
            pl.BlockSpec((1, 3, N), lambda b, i: (b, 0, 0)),
            pl.BlockSpec((1, TMQ, 3), lambda b, i: (b, i, 0)),
        ],
        out_specs=pl.BlockSpec((1, TMQ, K), lambda b, i: (b, i, 0)),
        out_shape=jax.ShapeDtypeStruct((nb, M, K), jnp.int32),
    )(pos, q_t)


@functools.cache
def _make_gather(rows):
    mesh = plsc.VectorSubcoreMesh(core_axis_name="c", subcore_axis_name="s")
    rpw = rows // NW
    nch = rpw // CHUNK

    @functools.partial(
        pl.kernel,
        mesh=mesh,
        out_type=jax.ShapeDtypeStruct((rows, C // 2), jnp.int32),
        scratch_types=[
            pltpu.VMEM((CHUNK,), jnp.int32),
            pltpu.VMEM((CHUNK, C // 2), jnp.int32),
            pltpu.SemaphoreType.DMA,
        ],
    )
    def gather(table_hbm, idx_hbm, out_hbm, idx_v, rows_v, sem):
        wid = lax.axis_index("s") * NC + lax.axis_index("c")

        def body(ch, carry):
            pltpu.sync_copy(idx_hbm.at[wid, ch], idx_v)
            pltpu.async_copy(table_hbm.at[idx_v], rows_v, sem).wait()
            pltpu.sync_copy(rows_v,
                            out_hbm.at[pl.ds(wid * rpw + ch * CHUNK, CHUNK)])
            return carry

        lax.fori_loop(0, nch, body, 0)

    return gather


def _gather(table, idx3):
    rows = idx3.shape[0] * idx3.shape[1] * idx3.shape[2]
    return _make_gather(rows)(table, idx3)


def _mlp_body(g_ref, q_ref, wp_ref, w1_ref, w2_ref, wo_ref,
              bin_ref, b1_ref, b2_ref, bo_ref, o_ref):
    q = q_ref[0]
    qc = lax.dot_general(q, wp_ref[...], (((1,), (1,)), ((), ())),
                         preferred_element_type=jnp.float32)
    qc = qc + bin_ref[...]
    g32 = g_ref[...]
    g_lo = lax.bitcast_convert_type(lax.shift_left(g32, 16), jnp.float32)
    g_hi = lax.bitcast_convert_type(g32 & jnp.int32(-65536), jnp.float32)
    g = jnp.concatenate([g_lo, g_hi], axis=1)
    x = g.reshape(TM, K, C) + qc[:, None, :]
    x = x.reshape(TM * K, C)
    h = lax.dot_general(jnp.maximum(x, 0.0), w1_ref[...],
                        (((1,), (1,)), ((), ())),
                        preferred_element_type=jnp.float32) + b1_ref[...]
    h = lax.dot_general(jnp.maximum(h, 0.0), w2_ref[...],
                        (((1,), (1,)), ((), ())),
                        preferred_element_type=jnp.float32) + b2_ref[...]
    y = jnp.max(h.reshape(TM, K, C), axis=1)
    o = lax.dot_general(y, wo_ref[...], (((1,), (1,)), ((), ())),
                        preferred_element_type=jnp.float32) + bo_ref[...]
    o_ref[0] = o


def _mlp(g, q_t, w_pos, w1, w2, w_out, b_in, b1, b2, b_out):
    nmt = M // TM
    nb = q_t.shape[0]
    return pl.pallas_call(
        _mlp_body,
        grid=(nb, nmt),
        in_specs=[
            pl.BlockSpec((TM * K, C // 2), lambda b, i: (b * nmt + i, 0)),
            pl.BlockSpec((1, TM, 3), lambda b, i: (b, i, 0)),
            pl.BlockSpec((C, 3), lambda b, i: (0, 0)),
            pl.BlockSpec((C, C), lambda b, i: (0, 0)),
            pl.BlockSpec((C, C), lambda b, i: (0, 0)),
            pl.BlockSpec((OUT, C), lambda b, i: (0, 0)),
            pl.BlockSpec((1, C), lambda b, i: (0, 0)),
            pl.BlockSpec((1, C), lambda b, i: (0, 0)),
            pl.BlockSpec((1, C), lambda b, i: (0, 0)),
            pl.BlockSpec((1, OUT), lambda b, i: (0, 0)),
        ],
        out_specs=pl.BlockSpec((1, TM, OUT), lambda b, i: (b, i, 0)),
        out_shape=jax.ShapeDtypeStruct((nb, M, OUT), jnp.float32),
    )(g, q_t, w_pos, w1, w2, w_out, b_in, b1, b2, b_out)


def kernel(pos, pos_non_manifold, latents, W_in, b_in, W1, b1, W2, b2,
           W_out, b_out):
    w_lat = W_in[:, :C]
    w_pos = W_in[:, C:]
    q_t = jnp.swapaxes(pos_non_manifold, 1, 2)

    outs = []
    for b in range(B):
        pos_b = pos[b:b + 1]
        q_b = q_t[b:b + 1]
        table_b = _build_table(latents[b:b + 1], pos_b, w_lat, w_pos)
        idx_b = _knn(pos_b, q_b)
        idx3_b = idx_b.reshape(NW, (M * K) // (NW * CHUNK), CHUNK)
        g_b = _gather(table_b, idx3_b)
        outs.append(_mlp(g_b, q_b, w_pos, W1, W2, W_out,
                         b_in.reshape(1, C), b1.reshape(1, C),
                         b2.reshape(1, C), b_out.reshape(1, OUT)))
    out_t = jnp.concatenate(outs, axis=0)
    return jnp.swapaxes(out_t, 1, 2)
